# Initial kernel scaffold; baseline (speedup 1.0000x reference)
import jax
import jax.numpy as jnp
from jax.experimental import pallas as pl


def kernel(segment_ids, table):
    raise NotImplementedError("write your pallas kernel here")


# speedup vs baseline: 3.0073x; 3.0073x over previous
import functools

import jax
import jax.numpy as jnp
from jax import lax
from jax.experimental import pallas as pl
from jax.experimental.pallas import tpu as pltpu
from jax.experimental.pallas import tpu_sc as plsc

EMBED = 64
NSEG = 3
L = 16
NC = 2
NS = 16
NW = NC * NS
CHUNK = 512


def _tec_body(ids_hbm, tab_hbm, out_hbm, ids_v, out_v, tab_v):
    wid = lax.axis_index("s") * NC + lax.axis_index("c")
    n_per_tile = ids_hbm.shape[0] // NW
    n_chunks = n_per_tile // CHUNK
    base = wid * n_per_tile

    pltpu.sync_copy(tab_hbm, tab_v)

    col = lax.iota(jnp.int32, L)

    def chunk_body(c, carry):
        off = base + c * CHUNK
        pltpu.sync_copy(ids_hbm.at[pl.ds(off, CHUNK)], ids_v)

        def group_body(g, carry2):
            gbase = g * L
            for j in range(L):
                bidx = jnp.full((L,), gbase + j, jnp.int32)
                id_b = plsc.load_gather(ids_v, [bidx])
                rowbase = id_b * EMBED
                for q in range(EMBED // L):
                    vals = plsc.load_gather(tab_v, [rowbase + (col + q * L)])
                    out_v[pl.ds((gbase + j) * EMBED + q * L, L)] = vals
            return carry2

        lax.fori_loop(0, CHUNK // L, group_body, 0, unroll=False)
        pltpu.sync_copy(out_v, out_hbm.at[pl.ds(off * EMBED, CHUNK * EMBED)])
        return carry

    lax.fori_loop(0, n_chunks, chunk_body, 0, unroll=False)


@functools.partial(jax.jit, static_argnames=("n",))
def _sc_lookup(ids_flat, tab_flat, n):
    mesh = plsc.VectorSubcoreMesh(core_axis_name="c", subcore_axis_name="s")
    kfn = pl.kernel(
        _tec_body,
        out_type=jax.ShapeDtypeStruct((n * EMBED,), jnp.float32),
        mesh=mesh,
        scratch_types=[
            pltpu.VMEM((CHUNK,), jnp.int32),
            pltpu.VMEM((CHUNK * EMBED,), jnp.float32),
            pltpu.VMEM((NSEG * EMBED,), jnp.float32),
        ],
        compiler_params=pltpu.CompilerParams(needs_layout_passes=False),
    )
    return kfn(ids_flat, tab_flat)


def kernel(segment_ids, table):
    b, s = segment_ids.shape
    n = b * s
    ids_flat = segment_ids.reshape(n).astype(jnp.int32)
    tab_flat = table.reshape(NSEG * EMBED)
    out = _sc_lookup(ids_flat, tab_flat, n)
    return out.reshape(b, s, EMBED)


# speedup vs baseline: 3.3005x; 1.0975x over previous
import functools

import jax
import jax.numpy as jnp
from jax import lax
from jax.experimental import pallas as pl
from jax.experimental.pallas import tpu as pltpu
from jax.experimental.pallas import tpu_sc as plsc

EMBED = 64
NSEG = 3
L = 16
NC = 2
NS = 16
NW = NC * NS
CHUNK = 800


def _build_rows(ids_v, out_v, tab_v, col):
    def group_body(g, carry):
        gbase = g * L
        for j in range(L):
            bidx = jnp.full((L,), gbase + j, jnp.int32)
            id_b = plsc.load_gather(ids_v, [bidx])
            rowbase = id_b * EMBED
            for q in range(EMBED // L):
                vals = plsc.load_gather(tab_v, [rowbase + (col + q * L)])
                out_v[pl.ds((gbase + j) * EMBED + q * L, L)] = vals
        return carry

    lax.fori_loop(0, CHUNK // L, group_body, 0, unroll=False)


def _tec_body(ids_hbm, tab_hbm, out_hbm,
              ids_v0, ids_v1, out_v0, out_v1, tab_v,
              isem0, isem1, osem0, osem1):
    wid = lax.axis_index("s") * NC + lax.axis_index("c")
    n_per_tile = ids_hbm.shape[0] // NW
    n_chunks = n_per_tile // CHUNK
    base = wid * n_per_tile

    pltpu.sync_copy(tab_hbm, tab_v)
    col = lax.iota(jnp.int32, L)

    bufs = ((ids_v0, out_v0, isem0, osem0), (ids_v1, out_v1, isem1, osem1))

    pltpu.async_copy(ids_hbm.at[pl.ds(base, CHUNK)], ids_v0, isem0)
    pltpu.async_copy(ids_hbm.at[pl.ds(base + CHUNK, CHUNK)], ids_v1, isem1)

    def pair_body(p, carry):
        for b, (ids_v, out_v, isem, osem) in enumerate(bufs):
            c = p * 2 + b
            off = base + c * CHUNK
            pltpu.make_async_copy(ids_hbm.at[pl.ds(base, CHUNK)], ids_v,
                                  isem).wait()
            @pl.when(c >= 2)
            def _wait_out():
                pltpu.make_async_copy(
                    out_v, out_hbm.at[pl.ds(base * EMBED, CHUNK * EMBED)],
                    osem).wait()

            _build_rows(ids_v, out_v, tab_v, col)

            pltpu.async_copy(
                out_v, out_hbm.at[pl.ds(off * EMBED, CHUNK * EMBED)], osem)

            @pl.when(c + 2 < n_chunks)
            def _prefetch():
                pltpu.async_copy(
                    ids_hbm.at[pl.ds(off + 2 * CHUNK, CHUNK)], ids_v, isem)
        return carry

    lax.fori_loop(0, n_chunks // 2, pair_body, 0, unroll=False)

    for b, (ids_v, out_v, isem, osem) in enumerate(bufs):
        pltpu.make_async_copy(
            out_v, out_hbm.at[pl.ds(base * EMBED, CHUNK * EMBED)],
            osem).wait()


@functools.partial(jax.jit, static_argnames=("n",))
def _sc_lookup(ids_flat, tab_flat, n):
    mesh = plsc.VectorSubcoreMesh(core_axis_name="c", subcore_axis_name="s")
    kfn = pl.kernel(
        _tec_body,
        out_type=jax.ShapeDtypeStruct((n * EMBED,), jnp.float32),
        mesh=mesh,
        scratch_types=[
            pltpu.VMEM((CHUNK,), jnp.int32),
            pltpu.VMEM((CHUNK,), jnp.int32),
            pltpu.VMEM((CHUNK * EMBED,), jnp.float32),
            pltpu.VMEM((CHUNK * EMBED,), jnp.float32),
            pltpu.VMEM((NSEG * EMBED,), jnp.float32),
            pltpu.SemaphoreType.DMA,
            pltpu.SemaphoreType.DMA,
            pltpu.SemaphoreType.DMA,
            pltpu.SemaphoreType.DMA,
        ],
        compiler_params=pltpu.CompilerParams(needs_layout_passes=False),
    )
    return kfn(ids_flat, tab_flat)


def kernel(segment_ids, table):
    b, s = segment_ids.shape
    n = b * s
    ids_flat = segment_ids.reshape(n).astype(jnp.int32)
    tab_flat = table.reshape(NSEG * EMBED)
    out = _sc_lookup(ids_flat, tab_flat, n)
    return out.reshape(b, s, EMBED)


# speedup vs baseline: 5.9106x; 1.7908x over previous
import functools

import jax
import jax.numpy as jnp
from jax import lax
from jax.experimental import pallas as pl
from jax.experimental.pallas import tpu as pltpu
from jax.experimental.pallas import tpu_sc as plsc

EMBED = 64
NSEG = 3
L = 16
NC = 2
NS = 16
NW = NC * NS
CHUNK = 800


def _build_rows(ids_v, out_v, tab_v):
    rows = [[tab_v[pl.ds(r * EMBED + q * L, L)] for q in range(EMBED // L)]
            for r in range(NSEG)]

    def group_body(g, carry):
        gbase = g * L
        ids16 = ids_v[pl.ds(gbase, L)]
        for j in range(L):
            b = jnp.take_along_axis(ids16, jnp.full((L,), j, jnp.int32),
                                    axis=0)
            m0 = b == 0
            m1 = b == 1
            obase = (gbase + j) * EMBED
            for q in range(EMBED // L):
                res = jnp.where(m1, rows[1][q],
                                jnp.where(m0, rows[0][q], rows[2][q]))
                out_v[pl.ds(obase + q * L, L)] = res
        return carry

    lax.fori_loop(0, CHUNK // L, group_body, 0, unroll=False)


def _tec_body(ids_hbm, tab_hbm, out_hbm,
              ids_v0, ids_v1, out_v0, out_v1, tab_v,
              isem0, isem1, osem0, osem1):
    wid = lax.axis_index("s") * NC + lax.axis_index("c")
    n_per_tile = ids_hbm.shape[0] // NW
    n_chunks = n_per_tile // CHUNK
    base = wid * n_per_tile

    pltpu.sync_copy(tab_hbm, tab_v)

    bufs = ((ids_v0, out_v0, isem0, osem0), (ids_v1, out_v1, isem1, osem1))

    pltpu.async_copy(ids_hbm.at[pl.ds(base, CHUNK)], ids_v0, isem0)
    pltpu.async_copy(ids_hbm.at[pl.ds(base + CHUNK, CHUNK)], ids_v1, isem1)

    def pair_body(p, carry):
        for b, (ids_v, out_v, isem, osem) in enumerate(bufs):
            c = p * 2 + b
            off = base + c * CHUNK
            pltpu.make_async_copy(ids_hbm.at[pl.ds(base, CHUNK)], ids_v,
                                  isem).wait()
            @pl.when(c >= 2)
            def _wait_out():
                pltpu.make_async_copy(
                    out_v, out_hbm.at[pl.ds(base * EMBED, CHUNK * EMBED)],
                    osem).wait()

            _build_rows(ids_v, out_v, tab_v)

            pltpu.async_copy(
                out_v, out_hbm.at[pl.ds(off * EMBED, CHUNK * EMBED)], osem)

            @pl.when(c + 2 < n_chunks)
            def _prefetch():
                pltpu.async_copy(
                    ids_hbm.at[pl.ds(off + 2 * CHUNK, CHUNK)], ids_v, isem)
        return carry

    lax.fori_loop(0, n_chunks // 2, pair_body, 0, unroll=False)

    for b, (ids_v, out_v, isem, osem) in enumerate(bufs):
        pltpu.make_async_copy(
            out_v, out_hbm.at[pl.ds(base * EMBED, CHUNK * EMBED)],
            osem).wait()


@functools.partial(jax.jit, static_argnames=("n",))
def _sc_lookup(ids_flat, tab_flat, n):
    mesh = plsc.VectorSubcoreMesh(core_axis_name="c", subcore_axis_name="s")
    kfn = pl.kernel(
        _tec_body,
        out_type=jax.ShapeDtypeStruct((n * EMBED,), jnp.float32),
        mesh=mesh,
        scratch_types=[
            pltpu.VMEM((CHUNK,), jnp.int32),
            pltpu.VMEM((CHUNK,), jnp.int32),
            pltpu.VMEM((CHUNK * EMBED,), jnp.float32),
            pltpu.VMEM((CHUNK * EMBED,), jnp.float32),
            pltpu.VMEM((NSEG * EMBED,), jnp.float32),
            pltpu.SemaphoreType.DMA,
            pltpu.SemaphoreType.DMA,
            pltpu.SemaphoreType.DMA,
            pltpu.SemaphoreType.DMA,
        ],
        compiler_params=pltpu.CompilerParams(needs_layout_passes=False),
    )
    return kfn(ids_flat, tab_flat)


def kernel(segment_ids, table):
    b, s = segment_ids.shape
    n = b * s
    ids_flat = segment_ids.reshape(n).astype(jnp.int32)
    tab_flat = table.reshape(NSEG * EMBED)
    out = _sc_lookup(ids_flat, tab_flat, n)
    return out.reshape(b, s, EMBED)


# speedup vs baseline: 10.3343x; 1.7484x over previous
import functools

import jax
import jax.numpy as jnp
from jax import lax
from jax.experimental import pallas as pl
from jax.experimental.pallas import tpu as pltpu
from jax.experimental.pallas import tpu_sc as plsc

EMBED = 64
NSEG = 3
L = 16
NC = 2
NS = 16
NW = NC * NS
CHUNK = 400


def _build_rows(ids_v, out_v, rows):
    def group_body(g, carry):
        gbase = g * L
        ids16 = ids_v[pl.ds(gbase, L)]
        for j in range(L):
            b = jnp.take_along_axis(ids16, jnp.full((L,), j, jnp.int32),
                                    axis=0)
            m0 = b == 0
            m1 = b == 1
            for q in range(EMBED // L):
                res = jnp.where(m1, rows[1][q],
                                jnp.where(m0, rows[0][q], rows[2][q]))
                out_v[gbase + j, pl.ds(q * L, L)] = res
        return carry

    lax.fori_loop(0, CHUNK // L, group_body, 0, unroll=False)


def _tec_body(ids_hbm, tab_hbm, out_hbm,
              ids_v0, ids_v1, out_v0, out_v1, tab_v,
              isem0, isem1, osem0, osem1):
    wid = lax.axis_index("s") * NC + lax.axis_index("c")
    n_per_tile = ids_hbm.shape[0] // NW
    n_chunks = n_per_tile // CHUNK
    base = wid * n_per_tile

    pltpu.sync_copy(tab_hbm, tab_v)
    rows = [[tab_v[pl.ds(r * EMBED + q * L, L)] for q in range(EMBED // L)]
            for r in range(NSEG)]

    bufs = ((ids_v0, out_v0, isem0, osem0), (ids_v1, out_v1, isem1, osem1))

    pltpu.async_copy(ids_hbm.at[pl.ds(base, CHUNK)], ids_v0, isem0)
    pltpu.async_copy(ids_hbm.at[pl.ds(base + CHUNK, CHUNK)], ids_v1, isem1)

    def pair_body(p, carry):
        for b, (ids_v, out_v, isem, osem) in enumerate(bufs):
            c = p * 2 + b
            off = base + c * CHUNK
            pltpu.make_async_copy(ids_hbm.at[pl.ds(base, CHUNK)], ids_v,
                                  isem).wait()
            @pl.when(c >= 2)
            def _wait_out():
                pltpu.make_async_copy(
                    out_v, out_hbm.at[pl.ds(base, CHUNK), :], osem).wait()

            _build_rows(ids_v, out_v, rows)

            pltpu.async_copy(out_v, out_hbm.at[pl.ds(off, CHUNK), :], osem)

            @pl.when(c + 2 < n_chunks)
            def _prefetch():
                pltpu.async_copy(
                    ids_hbm.at[pl.ds(off + 2 * CHUNK, CHUNK)], ids_v, isem)
        return carry

    lax.fori_loop(0, n_chunks // 2, pair_body, 0, unroll=False)

    for b, (ids_v, out_v, isem, osem) in enumerate(bufs):
        pltpu.make_async_copy(
            out_v, out_hbm.at[pl.ds(base, CHUNK), :], osem).wait()


@functools.partial(jax.jit, static_argnames=("n",))
def _sc_lookup(ids_flat, tab_flat, n):
    mesh = plsc.VectorSubcoreMesh(core_axis_name="c", subcore_axis_name="s")
    kfn = pl.kernel(
        _tec_body,
        out_type=jax.ShapeDtypeStruct((n, EMBED), jnp.float32),
        mesh=mesh,
        scratch_types=[
            pltpu.VMEM((CHUNK,), jnp.int32),
            pltpu.VMEM((CHUNK,), jnp.int32),
            pltpu.VMEM((CHUNK, EMBED), jnp.float32),
            pltpu.VMEM((CHUNK, EMBED), jnp.float32),
            pltpu.VMEM((NSEG * EMBED,), jnp.float32),
            pltpu.SemaphoreType.DMA,
            pltpu.SemaphoreType.DMA,
            pltpu.SemaphoreType.DMA,
            pltpu.SemaphoreType.DMA,
        ],
        compiler_params=pltpu.CompilerParams(needs_layout_passes=False),
    )
    return kfn(ids_flat, tab_flat)


def kernel(segment_ids, table):
    b, s = segment_ids.shape
    n = b * s
    ids_flat = segment_ids.reshape(n).astype(jnp.int32)
    tab_flat = table.reshape(NSEG * EMBED)
    out = _sc_lookup(ids_flat, tab_flat, n)
    return out.reshape(b, s, EMBED)
